# baseline (speedup 1.0000x reference)
import jax
import jax.numpy as jnp
from jax.experimental import pallas as pl
from jax.experimental.pallas import tpu as pltpu

X_POS = 5
CTX = 77
D = 512
BB = 128


def _body(ids_ref, table_ref, s_ref, tok_ref, out_ref, tok_out_ref,
          emb_scr, buf, gsem, osem):
    b = tok_out_ref.shape[0]
    nb = b // BB

    def gstart(k, c):
        pltpu.make_async_copy(
            table_ref.at[pl.ds(ids_ref[k], 1)],
            emb_scr.at[:, pl.ds(k * D, D)],
            gsem,
        ).start()
        return c

    jax.lax.fori_loop(0, CTX, gstart, 0)

    def gwait(k, c):
        pltpu.make_async_copy(
            table_ref.at[pl.ds(ids_ref[k], 1)],
            emb_scr.at[:, pl.ds(k * D, D)],
            gsem,
        ).wait()
        return c

    jax.lax.fori_loop(0, CTX, gwait, 0)

    buf[...] = jnp.broadcast_to(emb_scr[...], (BB, CTX * D))

    def copies(i, start):
        ops = (
            pltpu.make_async_copy(
                buf.at[:, pl.ds(0, X_POS * D)],
                out_ref.at[pl.ds(i * BB, BB), pl.ds(0, X_POS * D)],
                osem,
            ),
            pltpu.make_async_copy(
                s_ref.at[pl.ds(i * BB, BB)],
                out_ref.at[pl.ds(i * BB, BB), pl.ds(X_POS * D, D)],
                osem,
            ),
            pltpu.make_async_copy(
                buf.at[:, pl.ds((X_POS + 1) * D, (CTX - X_POS - 1) * D)],
                out_ref.at[pl.ds(i * BB, BB),
                           pl.ds((X_POS + 1) * D, (CTX - X_POS - 1) * D)],
                osem,
            ),
        )
        for op in ops:
            if start:
                op.start()
            else:
                op.wait()

    jax.lax.fori_loop(0, nb, lambda i, c: (copies(i, True), c)[1], 0)

    tok_out_ref[...] = jnp.broadcast_to(tok_ref[...], (b, CTX))

    jax.lax.fori_loop(0, nb, lambda i, c: (copies(i, False), c)[1], 0)


@jax.jit
def kernel(s_star, tokenized_composed, table):
    b = s_star.shape[0]
    ids = tokenized_composed.reshape(CTX)

    prompts_flat, tokenized = pl.pallas_call(
        _body,
        in_specs=[
            pl.BlockSpec(memory_space=pltpu.MemorySpace.SMEM),
            pl.BlockSpec(memory_space=pltpu.MemorySpace.HBM),
            pl.BlockSpec(memory_space=pltpu.MemorySpace.HBM),
            pl.BlockSpec((1, CTX), lambda: (0, 0)),
        ],
        out_specs=[
            pl.BlockSpec(memory_space=pltpu.MemorySpace.HBM),
            pl.BlockSpec((b, CTX), lambda: (0, 0)),
        ],
        out_shape=[
            jax.ShapeDtypeStruct((b, CTX * D), jnp.float32),
            jax.ShapeDtypeStruct((b, CTX), jnp.int32),
        ],
        scratch_shapes=[
            pltpu.VMEM((1, CTX * D), jnp.float32),
            pltpu.VMEM((BB, CTX * D), jnp.float32),
            pltpu.SemaphoreType.DMA,
            pltpu.SemaphoreType.DMA,
        ],
    )(ids, table, s_star.astype(jnp.float32), tokenized_composed)

    return (prompts_flat.reshape(b, CTX, D), tokenized)


# speedup vs baseline: 1.0501x; 1.0501x over previous
import jax
import jax.numpy as jnp
from jax.experimental import pallas as pl
from jax.experimental.pallas import tpu as pltpu

X_POS = 5
CTX = 77
D = 512
BB = 32
K = 8

_LO = X_POS * D
_HI = (CTX - X_POS - 1) * D


def _body(ids_ref, table_ref, s_ref, tok_ref, out_ref, tok_out_ref,
          emb_scr, svmem, gsem, ssem, osem, *bufs):
    b = tok_out_ref.shape[0]
    nb = b // BB

    s_stage = pltpu.make_async_copy(s_ref, svmem, ssem)
    s_stage.start()

    def gstart(k, c):
        pltpu.make_async_copy(
            table_ref.at[pl.ds(ids_ref[k], 1)],
            emb_scr.at[:, pl.ds(k * D, D)],
            gsem,
        ).start()
        return c

    jax.lax.fori_loop(0, CTX, gstart, 0)

    def gwait(k, c):
        pltpu.make_async_copy(
            table_ref.at[pl.ds(ids_ref[k], 1)],
            emb_scr.at[:, pl.ds(k * D, D)],
            gsem,
        ).wait()
        return c

    jax.lax.fori_loop(0, CTX, gwait, 0)

    emb = emb_scr[...]
    for k in range(K):
        bufs[k][...] = jnp.broadcast_to(emb, (BB, CTX * D))

    s_stage.wait()
    s_col = pltpu.make_async_copy(
        svmem, out_ref.at[:, pl.ds(_LO, D)], osem)
    s_col.start()

    def copies(j, start):
        for k in range(K):
            i = j * K + k
            ops = (
                pltpu.make_async_copy(
                    bufs[k].at[:, pl.ds(0, _LO)],
                    out_ref.at[pl.ds(i * BB, BB), pl.ds(0, _LO)],
                    osem,
                ),
                pltpu.make_async_copy(
                    bufs[k].at[:, pl.ds(_LO + D, _HI)],
                    out_ref.at[pl.ds(i * BB, BB), pl.ds(_LO + D, _HI)],
                    osem,
                ),
            )
            for op in ops:
                if start:
                    op.start()
                else:
                    op.wait()

    jax.lax.fori_loop(0, nb // K, lambda j, c: (copies(j, True), c)[1], 0)

    tok_out_ref[...] = jnp.broadcast_to(tok_ref[...], (b, CTX))

    jax.lax.fori_loop(0, nb // K, lambda j, c: (copies(j, False), c)[1], 0)
    s_col.wait()


@jax.jit
def kernel(s_star, tokenized_composed, table):
    b = s_star.shape[0]
    ids = tokenized_composed.reshape(CTX)

    prompts_flat, tokenized = pl.pallas_call(
        _body,
        in_specs=[
            pl.BlockSpec(memory_space=pltpu.MemorySpace.SMEM),
            pl.BlockSpec(memory_space=pltpu.MemorySpace.HBM),
            pl.BlockSpec(memory_space=pltpu.MemorySpace.HBM),
            pl.BlockSpec((1, CTX), lambda: (0, 0)),
        ],
        out_specs=[
            pl.BlockSpec(memory_space=pltpu.MemorySpace.HBM),
            pl.BlockSpec((b, CTX), lambda: (0, 0)),
        ],
        out_shape=[
            jax.ShapeDtypeStruct((b, CTX * D), jnp.float32),
            jax.ShapeDtypeStruct((b, CTX), jnp.int32),
        ],
        scratch_shapes=[
            pltpu.VMEM((1, CTX * D), jnp.float32),
            pltpu.VMEM((b, D), jnp.float32),
            pltpu.SemaphoreType.DMA,
            pltpu.SemaphoreType.DMA,
            pltpu.SemaphoreType.DMA,
        ] + [pltpu.VMEM((BB, CTX * D), jnp.float32) for _ in range(K)],
    )(ids, table, s_star.astype(jnp.float32), tokenized_composed)

    return (prompts_flat.reshape(b, CTX, D), tokenized)


# speedup vs baseline: 2.0445x; 1.9470x over previous
import jax
import jax.numpy as jnp
from jax.experimental import pallas as pl
from jax.experimental.pallas import tpu as pltpu

X_POS = 5
CTX = 77
D = 512
BB = 32
K = 8


def _body(ids_ref, table_ref, s_ref, tok_ref, out_ref, tok_out_ref,
          emb_scr, svmem, gsem, ssem, osem, *bufs):
    b = tok_out_ref.shape[0]
    nb = b // BB
    nj = nb // K

    s_stage = pltpu.make_async_copy(s_ref, svmem, ssem)
    s_stage.start()

    def gstart(k, c):
        pltpu.make_async_copy(
            table_ref.at[pl.ds(ids_ref[k], 1)],
            emb_scr.at[pl.ds(k, 1)],
            gsem,
        ).start()
        return c

    jax.lax.fori_loop(0, CTX, gstart, 0)

    def gwait(k, c):
        pltpu.make_async_copy(
            table_ref.at[pl.ds(ids_ref[k], 1)],
            emb_scr.at[pl.ds(k, 1)],
            gsem,
        ).wait()
        return c

    jax.lax.fori_loop(0, CTX, gwait, 0)

    emb = emb_scr[...]
    for k in range(K):
        bufs[k][...] = jnp.broadcast_to(emb[None], (BB, CTX, D))

    s_stage.wait()

    def tile_copy(k, j):
        return pltpu.make_async_copy(
            bufs[k], out_ref.at[pl.ds((j * K + k) * BB, BB)], osem)

    def step(j, c):
        for k in range(K):
            @pl.when(j > 0)
            def _wait_prev():
                tile_copy(k, j - 1).wait()

            i = j * K + k
            bufs[k][:, X_POS, :] = svmem[pl.ds(i * BB, BB), :]
            tile_copy(k, j).start()
        return c

    jax.lax.fori_loop(0, nj, step, 0)

    tok_out_ref[...] = jnp.broadcast_to(tok_ref[...], (b, CTX))

    for k in range(K):
        tile_copy(k, nj - 1).wait()


@jax.jit
def kernel(s_star, tokenized_composed, table):
    b = s_star.shape[0]
    ids = tokenized_composed.reshape(CTX)

    prompts, tokenized = pl.pallas_call(
        _body,
        in_specs=[
            pl.BlockSpec(memory_space=pltpu.MemorySpace.SMEM),
            pl.BlockSpec(memory_space=pltpu.MemorySpace.HBM),
            pl.BlockSpec(memory_space=pltpu.MemorySpace.HBM),
            pl.BlockSpec((1, CTX), lambda: (0, 0)),
        ],
        out_specs=[
            pl.BlockSpec(memory_space=pltpu.MemorySpace.HBM),
            pl.BlockSpec((b, CTX), lambda: (0, 0)),
        ],
        out_shape=[
            jax.ShapeDtypeStruct((b, CTX, D), jnp.float32),
            jax.ShapeDtypeStruct((b, CTX), jnp.int32),
        ],
        scratch_shapes=[
            pltpu.VMEM((CTX, D), jnp.float32),
            pltpu.VMEM((b, D), jnp.float32),
            pltpu.SemaphoreType.DMA,
            pltpu.SemaphoreType.DMA,
            pltpu.SemaphoreType.DMA,
        ] + [pltpu.VMEM((BB, CTX, D), jnp.float32) for _ in range(K)],
    )(ids, table, s_star.astype(jnp.float32), tokenized_composed)

    return (prompts, tokenized)


# speedup vs baseline: 2.0505x; 1.0029x over previous
import jax
import jax.numpy as jnp
from jax.experimental import pallas as pl
from jax.experimental.pallas import tpu as pltpu

X_POS = 5
CTX = 77
D = 512
BB = 32
K = 8


def _body(ids_ref, table_ref, s_ref, tok_ref, out_ref, tok_out_ref,
          emb_scr, svmem, gsem, ssem, osem, *bufs):
    b = tok_out_ref.shape[0]
    nb = b // BB
    nj = nb // K

    s_stage = pltpu.make_async_copy(s_ref, svmem, ssem)
    s_stage.start()

    def gstart(k, c):
        pltpu.make_async_copy(
            table_ref.at[pl.ds(ids_ref[k], 1)],
            emb_scr.at[pl.ds(k, 1)],
            gsem,
        ).start()
        return c

    jax.lax.fori_loop(0, CTX, gstart, 0)

    def gwait(k, c):
        pltpu.make_async_copy(
            table_ref.at[pl.ds(ids_ref[k], 1)],
            emb_scr.at[pl.ds(k, 1)],
            gsem,
        ).wait()
        return c

    jax.lax.fori_loop(0, CTX, gwait, 0)

    def tile_copy(k, j):
        return pltpu.make_async_copy(
            bufs[k], out_ref.at[pl.ds((j * K + k) * BB, BB)], osem)

    emb = emb_scr[...]
    s_stage.wait()
    for k in range(K):
        bufs[k][...] = jnp.broadcast_to(emb[None], (BB, CTX, D))
        bufs[k][:, X_POS, :] = svmem[pl.ds(k * BB, BB), :]
        tile_copy(k, 0).start()

    def step(j, c):
        for k in range(K):
            tile_copy(k, j - 1).wait()
            i = j * K + k
            bufs[k][:, X_POS, :] = svmem[pl.ds(i * BB, BB), :]
            tile_copy(k, j).start()
        return c

    jax.lax.fori_loop(1, nj, step, 0)

    tok_out_ref[...] = jnp.broadcast_to(tok_ref[...], (b, CTX))

    for k in range(K):
        tile_copy(k, nj - 1).wait()


@jax.jit
def kernel(s_star, tokenized_composed, table):
    b = s_star.shape[0]
    ids = tokenized_composed.reshape(CTX)

    prompts, tokenized = pl.pallas_call(
        _body,
        in_specs=[
            pl.BlockSpec(memory_space=pltpu.MemorySpace.SMEM),
            pl.BlockSpec(memory_space=pltpu.MemorySpace.HBM),
            pl.BlockSpec(memory_space=pltpu.MemorySpace.HBM),
            pl.BlockSpec((1, CTX), lambda: (0, 0)),
        ],
        out_specs=[
            pl.BlockSpec(memory_space=pltpu.MemorySpace.HBM),
            pl.BlockSpec((b, CTX), lambda: (0, 0)),
        ],
        out_shape=[
            jax.ShapeDtypeStruct((b, CTX, D), jnp.float32),
            jax.ShapeDtypeStruct((b, CTX), jnp.int32),
        ],
        scratch_shapes=[
            pltpu.VMEM((CTX, D), jnp.float32),
            pltpu.VMEM((b, D), jnp.float32),
            pltpu.SemaphoreType.DMA,
            pltpu.SemaphoreType.DMA,
            pltpu.SemaphoreType.DMA,
        ] + [pltpu.VMEM((BB, CTX, D), jnp.float32) for _ in range(K)],
    )(ids, table, s_star.astype(jnp.float32), tokenized_composed)

    return (prompts, tokenized)
